# Initial kernel scaffold; baseline (speedup 1.0000x reference)
import jax
import jax.numpy as jnp
from jax.experimental import pallas as pl


def kernel(x, edge_attr, params, edge_index):
    raise NotImplementedError("write your pallas kernel here")


# speedup vs baseline: 2.8999x; 2.8999x over previous
import functools

import jax
import jax.numpy as jnp
from jax import lax
from jax.experimental import pallas as pl
from jax.experimental.pallas import tpu as pltpu
from jax.experimental.pallas import tpu_sc as plsc

F32 = jnp.float32

N = 10000
NPAD = 10240
E = 320000
ROWS_PER_SUB = NPAD // 16

NC, NS = 2, 16
NW = NC * NS

CH = 512
CR = CH // 128
NCHUNK = E // CH
KMAX = (NCHUNK + NW - 1) // NW

CHS = 256
CRS = CHS // 128
NCHUNKS = E // CHS
KMAXS = (NCHUNKS + NW - 1) // NW

_SC_MESH = plsc.VectorSubcoreMesh(
    core_axis_name="c", subcore_axis_name="s", num_cores=NC, num_subcores=NS)


def _sds(shape, dtype=F32):
    return jax.ShapeDtypeStruct(shape, dtype)


@functools.partial(
    pl.kernel,
    out_type=(_sds((E, 64)), _sds((E, 64))),
    mesh=_SC_MESH,
    scratch_types=[
        pltpu.VMEM((CR, 128), jnp.int32),
        pltpu.VMEM((CR, 128), jnp.int32),
        pltpu.VMEM((CH, 64), F32),
        pltpu.VMEM((CH, 64), F32),
        pltpu.SemaphoreType.DMA,
    ],
    compiler_params=pltpu.CompilerParams(use_tc_tiling_on_sc=False),
)
def _sc_gather(src2d, dst2d, atab, btab, ago, bgo, idx_s, idx_d, rowsa, rowsb,
               sem):
    w = lax.axis_index("s") * NC + lax.axis_index("c")

    def body(k, carry):
        cid = w + k * NW

        @pl.when(cid < NCHUNK)
        def _():
            roff = pl.multiple_of(cid * CR, CR)
            eoff = pl.multiple_of(cid * CH, CH)
            pltpu.sync_copy(src2d.at[pl.ds(roff, CR)], idx_s)
            pltpu.sync_copy(dst2d.at[pl.ds(roff, CR)], idx_d)
            cps = []
            for j in range(CR):
                cps.append(pltpu.async_copy(
                    atab.at[idx_s.at[j]], rowsa.at[pl.ds(j * 128, 128)], sem))
                cps.append(pltpu.async_copy(
                    btab.at[idx_d.at[j]], rowsb.at[pl.ds(j * 128, 128)], sem))
            for cp in cps:
                cp.wait()
            pltpu.sync_copy(rowsa, ago.at[pl.ds(eoff, CH)])
            pltpu.sync_copy(rowsb, bgo.at[pl.ds(eoff, CH)])

        return carry

    lax.fori_loop(0, KMAX, body, 0)


@functools.partial(
    pl.kernel,
    out_type=(_sds((NC, NPAD, 128)), _sds((NC, NPAD, 16))),
    mesh=_SC_MESH,
    scratch_types=[
        pltpu.VMEM((CRS, 128), jnp.int32),
        pltpu.VMEM((CHS, 128), F32),
        pltpu.VMEM((128, 16), F32),
        pltpu.VMEM_SHARED((NPAD, 128), F32),
        pltpu.VMEM_SHARED((NPAD, 16), F32),
        pltpu.SemaphoreType.DMA,
    ],
    compiler_params=pltpu.CompilerParams(use_tc_tiling_on_sc=False),
)
def _sc_scatter(msg, dst2d, zrows, zcnt, ones, pout, cout, idx_d, rows, ones_v,
                acc, acc_cnt, sem):
    c = lax.axis_index("c")
    s = lax.axis_index("s")
    w = s * NC + c
    srow = pl.multiple_of(s * ROWS_PER_SUB, ROWS_PER_SUB)

    pltpu.sync_copy(zrows, acc.at[pl.ds(srow, ROWS_PER_SUB)])
    pltpu.sync_copy(zcnt, acc_cnt.at[pl.ds(srow, ROWS_PER_SUB)])
    pltpu.sync_copy(ones, ones_v)
    plsc.subcore_barrier()

    def body(k, carry):
        cid = w + k * NW

        @pl.when(cid < NCHUNKS)
        def _():
            roff = pl.multiple_of(cid * CRS, CRS)
            eoff = pl.multiple_of(cid * CHS, CHS)
            pltpu.sync_copy(dst2d.at[pl.ds(roff, CRS)], idx_d)
            pltpu.sync_copy(msg.at[pl.ds(eoff, CHS)], rows)
            cps = []
            for j in range(CRS):
                cps.append(pltpu.async_copy(
                    rows.at[pl.ds(j * 128, 128)], acc.at[idx_d.at[j]], sem,
                    add=True))
                cps.append(pltpu.async_copy(
                    ones_v, acc_cnt.at[idx_d.at[j]], sem, add=True))
            for cp in cps:
                cp.wait()

        return carry

    lax.fori_loop(0, KMAXS, body, 0)
    plsc.subcore_barrier()

    pltpu.sync_copy(acc.at[pl.ds(srow, ROWS_PER_SUB)],
                    pout.at[c, pl.ds(srow, ROWS_PER_SUB)])
    pltpu.sync_copy(acc_cnt.at[pl.ds(srow, ROWS_PER_SUB)],
                    cout.at[c, pl.ds(srow, ROWS_PER_SUB)])


_BEC = 8000


def _edgec_body(ea_ref, w_ref, b_ref, o0_ref, o1_ref, o2_ref):
    o = (jnp.dot(ea_ref[...], w_ref[...], preferred_element_type=F32)
         + b_ref[...])
    o0_ref[...] = o[:, 0:64]
    o1_ref[...] = o[:, 64:128]
    o2_ref[...] = o[:, 128:192]


def _edgec(edge_attr, w_all, b_all):
    out_spec = pl.BlockSpec((_BEC, 64), lambda i: (i, 0))
    return pl.pallas_call(
        _edgec_body,
        grid=(E // _BEC,),
        in_specs=[
            pl.BlockSpec((_BEC, 16), lambda i: (i, 0)),
            pl.BlockSpec((16, 192), lambda i: (0, 0)),
            pl.BlockSpec((1, 192), lambda i: (0, 0)),
        ],
        out_specs=(out_spec, out_spec, out_spec),
        out_shape=(_sds((E, 64)), _sds((E, 64)), _sds((E, 64))),
        compiler_params=pltpu.CompilerParams(
            dimension_semantics=("parallel",)),
    )(edge_attr, w_all, b_all)


_BE = 4000


def _mlp_body(ag_ref, bg_ref, c_ref, w2_ref, b2_ref, w3_ref, b3_ref, w4_ref,
              b4_ref, o_ref):
    h = jnp.maximum(ag_ref[...] + bg_ref[...] + c_ref[...], 0.0)
    h = jnp.maximum(
        jnp.dot(h, w2_ref[...], preferred_element_type=F32) + b2_ref[...], 0.0)
    h = jnp.maximum(
        jnp.dot(h, w3_ref[...], preferred_element_type=F32) + b3_ref[...], 0.0)
    o_ref[...] = (
        jnp.dot(h, w4_ref[...], preferred_element_type=F32) + b4_ref[...])


def _edge_mlp(ag, bg, c_l, w2, b2, w3, b3, w4, b4):
    small = lambda shape: pl.BlockSpec(shape, lambda i: (0, 0))
    return pl.pallas_call(
        _mlp_body,
        grid=(E // _BE,),
        in_specs=[
            pl.BlockSpec((_BE, 64), lambda i: (i, 0)),
            pl.BlockSpec((_BE, 64), lambda i: (i, 0)),
            pl.BlockSpec((_BE, 64), lambda i: (i, 0)),
            small((64, 64)), small((1, 64)),
            small((64, 64)), small((1, 64)),
            small((64, 128)), small((1, 128)),
        ],
        out_specs=pl.BlockSpec((_BE, 128), lambda i: (i, 0)),
        out_shape=_sds((E, 128)),
        compiler_params=pltpu.CompilerParams(
            dimension_semantics=("parallel",)),
    )(ag, bg, c_l, w2, b2, w3, b3, w4, b4)


def _np0_body(x_ref, wa_ref, wb_ref, a_ref, b_ref):
    x = x_ref[...]
    a_ref[...] = jnp.dot(x, wa_ref[...], preferred_element_type=F32)
    b_ref[...] = jnp.dot(x, wb_ref[...], preferred_element_type=F32)


def _np0(xpad, wa, wb):
    return pl.pallas_call(
        _np0_body,
        out_shape=(_sds((NPAD, 64)), _sds((NPAD, 64))),
    )(xpad, wa, wb)


def _inv_from_counts(cp_ref):
    cnt = cp_ref[0, :, 0:1] + cp_ref[1, :, 0:1]
    return 1.0 / jnp.maximum(cnt, 1.0)


def _comb0_body(p_ref, cp_ref, x_ref, wsh_ref, wsx_ref, wdh_ref, wdx_ref,
                h0_ref, a1_ref, b1_ref):
    h0 = (p_ref[0] + p_ref[1]) * _inv_from_counts(cp_ref)
    h0_ref[...] = h0
    x = x_ref[...]
    a1_ref[...] = (
        jnp.dot(h0, wsh_ref[...], preferred_element_type=F32)
        + jnp.dot(x, wsx_ref[...], preferred_element_type=F32))
    b1_ref[...] = (
        jnp.dot(h0, wdh_ref[...], preferred_element_type=F32)
        + jnp.dot(x, wdx_ref[...], preferred_element_type=F32))


def _comb0(p, cp, xpad, wsh, wsx, wdh, wdx):
    return pl.pallas_call(
        _comb0_body,
        out_shape=(_sds((NPAD, 128)), _sds((NPAD, 64)), _sds((NPAD, 64))),
    )(p, cp, xpad, wsh, wsx, wdh, wdx)


def _comb1_body(p_ref, cp_ref, h0_ref, wa_ref, wb_ref, an_ref, bn_ref):
    h1 = (p_ref[0] + p_ref[1]) * _inv_from_counts(cp_ref) + h0_ref[...]
    an_ref[...] = jnp.dot(h1, wa_ref[...], preferred_element_type=F32)
    bn_ref[...] = jnp.dot(h1, wb_ref[...], preferred_element_type=F32)


def _comb1(p, cp, h0, wa, wb):
    return pl.pallas_call(
        _comb1_body,
        out_shape=(_sds((NPAD, 64)), _sds((NPAD, 64))),
    )(p, cp, h0, wa, wb)


def _combn_body(p_ref, cp_ref, o_ref):
    o_ref[...] = (p_ref[0] + p_ref[1]) * _inv_from_counts(cp_ref)


def _combn(p, cp):
    return pl.pallas_call(
        _combn_body,
        out_shape=_sds((NPAD, 128)),
    )(p, cp)


def kernel(x, edge_attr, params, edge_index):
    src2d = edge_index[0].reshape(E // 128, 128)
    dst2d = edge_index[1].reshape(E // 128, 128)
    xpad = jnp.pad(x, ((0, NPAD - N), (0, 0)))

    p0, p1, pn = params["layer0"], params["layer1"], params["layern"]
    w0, b0 = p0[0]
    w1, b1 = p1[0]
    wn, bn = pn[0]

    w0a, w0b, w0c = w0[:128], w0[128:256], w0[256:272]
    w1sh, w1sx = w1[:128], w1[128:256]
    w1dh, w1dx = w1[256:384], w1[384:512]
    w1c = w1[512:528]
    wna, wnb, wnc = wn[:128], wn[128:256], wn[256:272]

    wc_all = jnp.concatenate([w0c, w1c, wnc], axis=1)
    b_all = jnp.concatenate([b0, b1, bn]).reshape(1, 192)

    mlp_tail = lambda p: sum(((w, b.reshape(1, -1)) for w, b in p[1:]), ())

    zrows = jnp.zeros((ROWS_PER_SUB, 128), F32)
    zcnt = jnp.zeros((ROWS_PER_SUB, 16), F32)
    ones = jnp.ones((128, 16), F32)

    c0, c1, cn = _edgec(edge_attr, wc_all, b_all)

    a0, bb0 = _np0(xpad, w0a, w0b)
    ag, bg = _sc_gather(src2d, dst2d, a0, bb0)
    msg = _edge_mlp(ag, bg, c0, *mlp_tail(p0))
    pp0, cp = _sc_scatter(msg, dst2d, zrows, zcnt, ones)

    h0, a1, b1t = _comb0(pp0, cp, xpad, w1sh, w1sx, w1dh, w1dx)
    ag, bg = _sc_gather(src2d, dst2d, a1, b1t)
    msg = _edge_mlp(ag, bg, c1, *mlp_tail(p1))
    pp1, _ = _sc_scatter(msg, dst2d, zrows, zcnt, ones)

    an, bn_t = _comb1(pp1, cp, h0, wna, wnb)
    ag, bg = _sc_gather(src2d, dst2d, an, bn_t)
    msg = _edge_mlp(ag, bg, cn, *mlp_tail(pn))
    ppn, _ = _sc_scatter(msg, dst2d, zrows, zcnt, ones)

    out = _combn(ppn, cp)
    return out[:N]
